# Initial kernel scaffold; baseline (speedup 1.0000x reference)
import jax
import jax.numpy as jnp
from jax.experimental import pallas as pl


def kernel(verts, faces):
    raise NotImplementedError("write your pallas kernel here")


# speedup vs baseline: 15.1946x; 15.1946x over previous
import functools

import jax
import jax.numpy as jnp
from jax import lax
from jax.experimental import pallas as pl
from jax.experimental.pallas import tpu as pltpu
from jax.experimental.pallas import tpu_sc as plsc

NC = 2
NS = 16
CH = 128
CPB = 16
NBLK = 37
PT = NBLK * CPB * CH
NVSC = 100096
RPT = NVSC // NS
W = 8


def _sc_accumulate(v4flat, zeros4, dst_i, src_i):
    mesh = plsc.VectorSubcoreMesh(core_axis_name="c", subcore_axis_name="s")

    @functools.partial(
        pl.kernel,
        out_type=jax.ShapeDtypeStruct((NC, NVSC, W), jnp.float32),
        mesh=mesh,
        scratch_types=[
            pltpu.VMEM_SHARED((NVSC, W), jnp.float32),
            pltpu.VMEM((CPB, CH), jnp.int32),
            pltpu.VMEM((CPB, CH), jnp.int32),
            pltpu.VMEM((CH, W), jnp.float32),
            pltpu.SemaphoreType.DMA,
        ],
        compiler_params=pltpu.CompilerParams(use_tc_tiling_on_sc=False),
    )
    def k(v4_hbm, z_hbm, dsti_hbm, srci_hbm, acc_hbm, acc, dsti, srci, rows,
          sem):
        c = lax.axis_index("c")
        s = lax.axis_index("s")
        base = s * RPT
        pltpu.sync_copy(z_hbm.at[c, pl.ds(base, RPT)], acc.at[pl.ds(base, RPT)])
        plsc.subcore_barrier()

        def blk_body(b, carry):
            pltpu.sync_copy(dsti_hbm.at[c, s, pl.ds(b * CPB, CPB)], dsti)
            pltpu.sync_copy(srci_hbm.at[c, s, pl.ds(b * CPB, CPB)], srci)

            def ch_body(j, carry2):
                pltpu.async_copy(v4_hbm.at[srci.at[j]], rows, sem).wait()
                pltpu.sync_copy(rows, acc.at[dsti.at[j]], add=True)
                return carry2

            lax.fori_loop(0, CPB, ch_body, 0)
            return carry

        lax.fori_loop(0, NBLK, blk_body, 0)
        plsc.subcore_barrier()
        pltpu.sync_copy(acc.at[pl.ds(base, RPT)], acc_hbm.at[c, pl.ds(base, RPT)])

    return k(v4flat, zeros4, dst_i, src_i)


def _tc_normalize(acc2, v4flat):
    rows = acc2.shape[0]
    br = 512
    assert rows % br == 0

    def body(a_ref, v_ref, o_ref):
        a = a_ref[...]
        v = v_ref[...]
        d = a[:, 3:4]
        o_ref[...] = (d * v - a) / (d + 1e-12)

    return pl.pallas_call(
        body,
        grid=(rows // br,),
        in_specs=[
            pl.BlockSpec((br, W), lambda i: (i, 0)),
            pl.BlockSpec((br, W), lambda i: (i, 0)),
        ],
        out_specs=pl.BlockSpec((br, W), lambda i: (i, 0)),
        out_shape=jax.ShapeDtypeStruct((rows, W), jnp.float32),
    )(acc2, v4flat)


def kernel(verts, faces):
    b, nv, _ = verts.shape
    nf = faces.shape[1]
    nvsc_real = (b // NC) * nv

    v = verts.reshape(b * nv, 3)
    v4 = jnp.concatenate(
        [v, jnp.ones((b * nv, 1), v.dtype), jnp.zeros((b * nv, W - 4), v.dtype)],
        axis=1)
    v4 = v4.reshape(NC, nvsc_real, W)
    v4 = jnp.pad(v4, ((0, 0), (0, NVSC - nvsc_real), (0, 0)))
    v4flat = v4.reshape(NC * NVSC, W)

    ft = jnp.transpose(faces, (2, 0, 1))
    loc = ((jnp.arange(b, dtype=faces.dtype) % 2) * nv).reshape(1, b, 1)
    ft = ft + loc
    fa, fb, fc = ft[0], ft[1], ft[2]
    dst = jnp.stack([fa, fb, fc, fb, fc, fa], axis=1).reshape(NC, 12 * nf)
    src = jnp.stack([fb, fc, fa, fa, fb, fc], axis=1).reshape(NC, 12 * nf)
    padn = NS * PT - dst.shape[1]
    fill = jnp.full((NC, padn), nvsc_real, faces.dtype)
    dst = jnp.concatenate([dst, fill], axis=1)
    src = jnp.concatenate([src, fill], axis=1)
    src = src + (jnp.arange(NC, dtype=src.dtype) * NVSC).reshape(NC, 1)
    dst_i = dst.reshape(NC, NS, NBLK * CPB, CH)
    src_i = src.reshape(NC, NS, NBLK * CPB, CH)

    zeros4 = jnp.zeros((NC, NVSC, W), jnp.float32)
    acc = _sc_accumulate(v4flat, zeros4, dst_i, src_i)
    out4 = _tc_normalize(acc.reshape(NC * NVSC, W), v4flat)
    out4 = out4.reshape(NC, NVSC, W)[:, :nvsc_real, :3]
    return out4.reshape(b, nv, 3)


# speedup vs baseline: 18.4269x; 1.2127x over previous
import functools

import jax
import jax.numpy as jnp
from jax import lax
from jax.experimental import pallas as pl
from jax.experimental.pallas import tpu as pltpu
from jax.experimental.pallas import tpu_sc as plsc

NC = 2
NS = 16
CH = 128
CPB = 16
NBLK = 37
PT = NBLK * CPB * CH
NVSC = 100096
RPT = NVSC // NS
W = 8


def _sc_accumulate(v4flat, zeros4, dst_i, src_i):
    mesh = plsc.VectorSubcoreMesh(core_axis_name="c", subcore_axis_name="s")

    @functools.partial(
        pl.kernel,
        out_type=jax.ShapeDtypeStruct((NC, NVSC, W), jnp.float32),
        mesh=mesh,
        scratch_types=[
            pltpu.VMEM_SHARED((NVSC, W), jnp.float32),
            pltpu.VMEM((CPB, CH), jnp.int32),
            pltpu.VMEM((CPB, CH), jnp.int32),
            pltpu.VMEM((CH, W), jnp.float32),
            pltpu.VMEM((CH, W), jnp.float32),
            pltpu.SemaphoreType.DMA,
            pltpu.SemaphoreType.DMA,
        ],
        compiler_params=pltpu.CompilerParams(use_tc_tiling_on_sc=False),
    )
    def k(v4_hbm, z_hbm, dsti_hbm, srci_hbm, acc_hbm, acc, dsti, srci,
          rows_a, rows_b, sem_a, sem_b):
        c = lax.axis_index("c")
        s = lax.axis_index("s")
        base = s * RPT
        pltpu.sync_copy(z_hbm.at[c, pl.ds(base, RPT)], acc.at[pl.ds(base, RPT)])
        plsc.subcore_barrier()

        def blk_body(b, carry):
            pltpu.sync_copy(dsti_hbm.at[c, s, pl.ds(b * CPB, CPB)], dsti)
            pltpu.sync_copy(srci_hbm.at[c, s, pl.ds(b * CPB, CPB)], srci)
            pltpu.async_copy(v4_hbm.at[srci.at[0]], rows_a, sem_a)

            def pair_body(p, carry2):
                j = 2 * p
                pltpu.async_copy(v4_hbm.at[srci.at[j + 1]], rows_b, sem_b)
                pltpu.make_async_copy(v4_hbm.at[srci.at[j]], rows_a,
                                      sem_a).wait()
                pltpu.sync_copy(rows_a, acc.at[dsti.at[j]], add=True)

                @pl.when(j + 2 < CPB)
                def _():
                    pltpu.async_copy(v4_hbm.at[srci.at[j + 2]], rows_a, sem_a)

                pltpu.make_async_copy(v4_hbm.at[srci.at[j + 1]], rows_b,
                                      sem_b).wait()
                pltpu.sync_copy(rows_b, acc.at[dsti.at[j + 1]], add=True)
                return carry2

            lax.fori_loop(0, CPB // 2, pair_body, 0)
            return carry

        lax.fori_loop(0, NBLK, blk_body, 0)
        plsc.subcore_barrier()
        pltpu.sync_copy(acc.at[pl.ds(base, RPT)], acc_hbm.at[c, pl.ds(base, RPT)])

    return k(v4flat, zeros4, dst_i, src_i)


def _tc_normalize(acc2, v4flat):
    rows = acc2.shape[0]
    br = 512
    assert rows % br == 0

    def body(a_ref, v_ref, o_ref):
        a = a_ref[...]
        v = v_ref[...]
        d = a[:, 3:4]
        o_ref[...] = (d * v - a) / (d + 1e-12)

    return pl.pallas_call(
        body,
        grid=(rows // br,),
        in_specs=[
            pl.BlockSpec((br, W), lambda i: (i, 0)),
            pl.BlockSpec((br, W), lambda i: (i, 0)),
        ],
        out_specs=pl.BlockSpec((br, W), lambda i: (i, 0)),
        out_shape=jax.ShapeDtypeStruct((rows, W), jnp.float32),
    )(acc2, v4flat)


def kernel(verts, faces):
    b, nv, _ = verts.shape
    nf = faces.shape[1]
    nvsc_real = (b // NC) * nv

    v = verts.reshape(b * nv, 3)
    v4 = jnp.concatenate(
        [v, jnp.ones((b * nv, 1), v.dtype), jnp.zeros((b * nv, W - 4), v.dtype)],
        axis=1)
    v4 = v4.reshape(NC, nvsc_real, W)
    v4 = jnp.pad(v4, ((0, 0), (0, NVSC - nvsc_real), (0, 0)))
    v4flat = v4.reshape(NC * NVSC, W)

    ft = jnp.transpose(faces, (2, 0, 1))
    loc = ((jnp.arange(b, dtype=faces.dtype) % 2) * nv).reshape(1, b, 1)
    ft = ft + loc
    fa, fb, fc = ft[0], ft[1], ft[2]
    dst = jnp.stack([fa, fb, fc, fb, fc, fa], axis=1).reshape(NC, 12 * nf)
    src = jnp.stack([fb, fc, fa, fa, fb, fc], axis=1).reshape(NC, 12 * nf)
    padn = NS * PT - dst.shape[1]
    fill = jnp.full((NC, padn), nvsc_real, faces.dtype)
    dst = jnp.concatenate([dst, fill], axis=1)
    src = jnp.concatenate([src, fill], axis=1)
    src = src + (jnp.arange(NC, dtype=src.dtype) * NVSC).reshape(NC, 1)
    dst_i = dst.reshape(NC, NS, NBLK * CPB, CH)
    src_i = src.reshape(NC, NS, NBLK * CPB, CH)

    zeros4 = jnp.zeros((NC, NVSC, W), jnp.float32)
    acc = _sc_accumulate(v4flat, zeros4, dst_i, src_i)
    out4 = _tc_normalize(acc.reshape(NC * NVSC, W), v4flat)
    out4 = out4.reshape(NC, NVSC, W)[:, :nvsc_real, :3]
    return out4.reshape(b, nv, 3)
